# baseline (speedup 1.0000x reference)
import functools

import jax
import jax.numpy as jnp
from jax import lax
from jax.experimental import pallas as pl
from jax.experimental.pallas import tpu as pltpu
from jax.experimental.pallas import tpu_sc as plsc


def _make_sc_pair_gather2(t, dd):
    info = plsc.get_sparse_core_info()
    nw = info.num_cores * info.num_subcores
    tpw = t // nw
    chunk = 80
    assert t % nw == 0 and tpw % chunk == 0
    n_chunks = tpw // chunk

    mesh = plsc.VectorSubcoreMesh(core_axis_name="c", subcore_axis_name="s")

    @functools.partial(
        pl.kernel,
        mesh=mesh,
        out_type=[
            jax.ShapeDtypeStruct((t, dd), jnp.float32),
            jax.ShapeDtypeStruct((t, dd), jnp.float32),
        ],
        scratch_types=[
            pltpu.VMEM((tpw,), jnp.int32),
            pltpu.VMEM((chunk, dd), jnp.float32),
            pltpu.VMEM((chunk, dd), jnp.float32),
            pltpu.SemaphoreType.DMA,
            pltpu.SemaphoreType.DMA,
        ],
    )
    def gather(ta, ia, tb, ib, out_a, out_b, idx_v, r0, r1, s0, s1):
        wid = lax.axis_index("s") * info.num_cores + lax.axis_index("c")
        base = wid * tpw
        rows = (r0, r1)
        sems = (s0, s1)

        def run(table_hbm, idx_hbm, out_hbm):
            pltpu.sync_copy(idx_hbm.at[pl.ds(base, tpw)], idx_v)

            def fire(j):
                pltpu.async_copy(
                    table_hbm.at[idx_v.at[pl.ds(j * chunk, chunk)]],
                    rows[j % 2],
                    sems[j % 2],
                )

            fire(0)
            for j in range(n_chunks):
                if j + 1 < n_chunks:
                    fire(j + 1)
                pltpu.make_async_copy(
                    table_hbm.at[idx_v.at[pl.ds(j * chunk, chunk)]],
                    rows[j % 2],
                    sems[j % 2],
                ).wait()
                pltpu.sync_copy(
                    rows[j % 2], out_hbm.at[pl.ds(base + j * chunk, chunk)]
                )

        run(tb, ib, out_b)
        run(ta, ia, out_a)

    return gather


def _make_tc_logits(t, d, n, blk=1024):
    assert t % blk == 0

    def body(te_ref, log_ref, ol_ref):
        ol_ref[...] = jnp.dot(
            te_ref[...].astype(jnp.bfloat16), log_ref[...],
            preferred_element_type=jnp.float32,
        )

    return pl.pallas_call(
        body,
        grid=(t // blk,),
        in_specs=[
            pl.BlockSpec((blk, d), lambda i: (i, 0)),
            pl.BlockSpec((d, n), lambda i: (0, 0)),
        ],
        out_specs=pl.BlockSpec((blk, n), lambda i: (i, 0)),
        out_shape=jax.ShapeDtypeStruct((t, n), jnp.float32),
    )


def kernel(source_enumerate, target_enumerate, src_embs, tgt_embs, logits):
    b, s = source_enumerate.shape
    t = b * s
    src_v, d = src_embs.shape
    tgt_v = tgt_embs.shape[0]
    n = logits.shape[1]

    src_idx = source_enumerate.reshape(t).astype(jnp.int32)
    tgt_idx = target_enumerate.reshape(t).astype(jnp.int32)
    table2_src = src_embs.reshape(src_v // 2, 2 * d)
    table2_tgt = tgt_embs.reshape(tgt_v // 2, 2 * d)
    pairs_src, pairs_tgt = _make_sc_pair_gather2(t, 2 * d)(
        table2_src, src_idx >> 1, table2_tgt, tgt_idx >> 1
    )

    odd_s = (src_idx & 1)[:, None].astype(jnp.bool_)
    odd_t = (tgt_idx & 1)[:, None].astype(jnp.bool_)
    src_emb = jnp.where(odd_s, pairs_src[:, d:], pairs_src[:, :d])
    te = jnp.where(odd_t, pairs_tgt[:, d:], pairs_tgt[:, :d])

    log_bf = logits.astype(jnp.bfloat16)
    ol = _make_tc_logits(t, d, n)(te, log_bf)

    return (
        src_emb.reshape(b, s, d),
        te.reshape(b, s, d),
        ol.reshape(b, s, n),
    )


# speedup vs baseline: 1.0150x; 1.0150x over previous
import functools

import jax
import jax.numpy as jnp
from jax import lax
from jax.experimental import pallas as pl
from jax.experimental.pallas import tpu as pltpu
from jax.experimental.pallas import tpu_sc as plsc


def _make_sc_pair_gather2(t, dd):
    info = plsc.get_sparse_core_info()
    nw = info.num_cores * info.num_subcores
    tpw = t // nw
    chunk = 80
    assert t % nw == 0 and tpw % chunk == 0
    n_chunks = tpw // chunk

    mesh = plsc.VectorSubcoreMesh(core_axis_name="c", subcore_axis_name="s")

    @functools.partial(
        pl.kernel,
        mesh=mesh,
        out_type=jax.ShapeDtypeStruct((t, dd), jnp.float32),
        scratch_types=[
            pltpu.VMEM((tpw,), jnp.int32),
            pltpu.VMEM((chunk, dd), jnp.float32),
            pltpu.VMEM((chunk, dd), jnp.float32),
            pltpu.SemaphoreType.DMA,
            pltpu.SemaphoreType.DMA,
        ],
    )
    def gather(table_hbm, idx_hbm, out_hbm, idx_v, r0, r1, s0, s1):
        wid = lax.axis_index("s") * info.num_cores + lax.axis_index("c")
        base = wid * tpw
        rows = (r0, r1)
        sems = (s0, s1)
        pltpu.sync_copy(idx_hbm.at[pl.ds(base, tpw)], idx_v)

        def fire(j):
            pltpu.async_copy(
                table_hbm.at[idx_v.at[pl.ds(j * chunk, chunk)]],
                rows[j % 2],
                sems[j % 2],
            )

        fire(0)
        for j in range(n_chunks):
            if j + 1 < n_chunks:
                fire(j + 1)
            pltpu.make_async_copy(
                table_hbm.at[idx_v.at[pl.ds(j * chunk, chunk)]],
                rows[j % 2],
                sems[j % 2],
            ).wait()
            pltpu.sync_copy(
                rows[j % 2], out_hbm.at[pl.ds(base + j * chunk, chunk)]
            )

    return gather


def _make_tc_logits(t, d, n, blk=1024):
    assert t % blk == 0

    def body(te_ref, log_ref, ol_ref):
        ol_ref[...] = jnp.dot(
            te_ref[...].astype(jnp.bfloat16), log_ref[...],
            preferred_element_type=jnp.float32,
        )

    return pl.pallas_call(
        body,
        grid=(t // blk,),
        in_specs=[
            pl.BlockSpec((blk, d), lambda i: (i, 0)),
            pl.BlockSpec((d, n), lambda i: (0, 0)),
        ],
        out_specs=pl.BlockSpec((blk, n), lambda i: (i, 0)),
        out_shape=jax.ShapeDtypeStruct((t, n), jnp.float32),
    )


def kernel(source_enumerate, target_enumerate, src_embs, tgt_embs, logits):
    b, s = source_enumerate.shape
    t = b * s
    src_v, d = src_embs.shape
    tgt_v = tgt_embs.shape[0]
    n = logits.shape[1]

    src_idx = source_enumerate.reshape(t).astype(jnp.int32)
    tgt_idx = target_enumerate.reshape(t).astype(jnp.int32)
    table2_src = src_embs.reshape(src_v // 2, 2 * d)
    table2_tgt = tgt_embs.reshape(tgt_v // 2, 2 * d)
    sc_gather = _make_sc_pair_gather2(t, 2 * d)
    pairs_tgt = sc_gather(table2_tgt, tgt_idx >> 1)
    pairs_src = sc_gather(table2_src, src_idx >> 1)

    odd_s = (src_idx & 1)[:, None].astype(jnp.bool_)
    odd_t = (tgt_idx & 1)[:, None].astype(jnp.bool_)
    src_emb = jnp.where(odd_s, pairs_src[:, d:], pairs_src[:, :d])
    te = jnp.where(odd_t, pairs_tgt[:, d:], pairs_tgt[:, :d])

    log_bf = logits.astype(jnp.bfloat16)
    ol = _make_tc_logits(t, d, n)(te, log_bf)

    return (
        src_emb.reshape(b, s, d),
        te.reshape(b, s, d),
        ol.reshape(b, s, n),
    )
